# Initial kernel scaffold; baseline (speedup 1.0000x reference)
import jax
import jax.numpy as jnp
from jax.experimental import pallas as pl


def kernel(seg_embeddings, edge_index, edge_weights, traj_ids, traj_lengths):
    raise NotImplementedError("write your pallas kernel here")


# speedup vs baseline: 1.2182x; 1.2182x over previous
import functools

import jax
import jax.numpy as jnp
from jax import lax
from jax.experimental import pallas as pl
from jax.experimental.pallas import tpu as pltpu
from jax.experimental.pallas import tpu_sc as plsc

N_SEG, D = 50000, 512
B, L = 16, 2048
ROWS = B * L
NW = 32
RPW = ROWS // NW
C = 128
NCHUNK = RPW // C
ZROWS = 64
VPR = D // 16


def _body(table_hbm, ids_hbm, len_hbm, out_hbm, idx_v, rows_v, zbuf, len_v, sem):
    nc = 2
    wid = lax.axis_index("s") * nc + lax.axis_index("c")
    base = wid * RPW

    pltpu.sync_copy(len_hbm, len_v.at[pl.ds(0, 16)])
    pltpu.sync_copy(ids_hbm.at[pl.ds(base, RPW)], idx_v)

    b = wid >> 1
    off = (wid & 1) * RPW
    mylen = len_v[pl.ds(b, 16)][0]
    nv = jnp.clip(mylen - off, 0, RPW)

    def _zrow(r, _):
        def _zcol(j, _):
            zbuf[r, pl.ds(j * 16, 16)] = jnp.zeros((16,), jnp.float32)
            return 0
        return lax.fori_loop(0, VPR, _zcol, 0)
    lax.fori_loop(0, ZROWS, _zrow, 0)

    for i in range(NCHUNK):
        s = i * C

        @pl.when(s + C <= nv)
        def _full():
            pltpu.async_copy(
                table_hbm.at[idx_v.at[pl.ds(s, C)]], rows_v, sem).wait()
            pltpu.sync_copy(rows_v, out_hbm.at[pl.ds(base + s, C)])

        @pl.when(nv <= s)
        def _empty():
            pltpu.sync_copy(zbuf, out_hbm.at[pl.ds(base + s, ZROWS)])
            pltpu.sync_copy(zbuf, out_hbm.at[pl.ds(base + s + ZROWS, ZROWS)])

        @pl.when(jnp.logical_and(s < nv, nv < s + C))
        def _mixed():
            pltpu.async_copy(
                table_hbm.at[idx_v.at[pl.ds(s, C)]], rows_v, sem).wait()

            def _ztail(k, _):
                r = lax.shift_right_logical(k, 5)
                col = jnp.bitwise_and(k, VPR - 1) * 16
                rows_v[r, pl.ds(col, 16)] = jnp.zeros((16,), jnp.float32)
                return 0
            lax.fori_loop((nv - s) * VPR, C * VPR, _ztail, 0)
            pltpu.sync_copy(rows_v, out_hbm.at[pl.ds(base + s, C)])


@jax.jit
def _sembed(table, ids_flat, lengths):
    mesh = plsc.VectorSubcoreMesh(core_axis_name="c", subcore_axis_name="s")
    f = functools.partial(
        pl.kernel,
        out_type=jax.ShapeDtypeStruct((ROWS, D), jnp.float32),
        mesh=mesh,
        scratch_types=[
            pltpu.VMEM((RPW,), jnp.int32),
            pltpu.VMEM((C, D), jnp.float32),
            pltpu.VMEM((ZROWS, D), jnp.float32),
            pltpu.VMEM((32,), jnp.int32),
            pltpu.SemaphoreType.DMA,
        ],
    )(_body)
    return f(table, ids_flat, lengths)


def kernel(seg_embeddings, edge_index, edge_weights, traj_ids, traj_lengths):
    del edge_index, edge_weights
    out = _sembed(seg_embeddings, traj_ids.reshape(ROWS), traj_lengths)
    return out.reshape(B, L, D)


# speedup vs baseline: 1.5852x; 1.3012x over previous
import functools

import jax
import jax.numpy as jnp
from jax import lax
from jax.experimental import pallas as pl
from jax.experimental.pallas import tpu as pltpu
from jax.experimental.pallas import tpu_sc as plsc

N_SEG, D = 50000, 512
B, L = 16, 2048
ROWS = B * L
NW = 32
RPW = ROWS // NW
C = 64
NCHUNK = RPW // C
VPR = D // 16


def _body(table_hbm, ids_hbm, len_hbm, out_hbm,
          idx_v, rows0, rows1, zbuf, len_v, gsem, wsem0, wsem1):
    nc = 2
    wid = lax.axis_index("s") * nc + lax.axis_index("c")
    base = wid * RPW

    pltpu.sync_copy(len_hbm, len_v.at[pl.ds(0, 16)])
    pltpu.sync_copy(ids_hbm.at[pl.ds(base, RPW)], idx_v)

    b = wid >> 1
    off = (wid & 1) * RPW
    mylen = len_v[pl.ds(b, 16)][0]
    nv = jnp.clip(mylen - off, 0, RPW)

    rows = (rows0, rows1)
    wsems = (wsem0, wsem1)

    def issue_gather(i):
        s = i * C

        @pl.when(s < nv)
        def _():
            pltpu.async_copy(
                table_hbm.at[idx_v.at[pl.ds(s, C)]], rows[i % 2], gsem)

    issue_gather(0)

    def _zrow(r, _):
        def _zcol(j, _):
            zbuf[r, pl.ds(j * 16, 16)] = jnp.zeros((16,), jnp.float32)
            return 0
        return lax.fori_loop(0, VPR, _zcol, 0)
    lax.fori_loop(0, C, _zrow, 0)

    for i in range(1, NCHUNK + 1):
        if i < NCHUNK:
            if i >= 2:
                pltpu.make_async_copy(
                    rows[i % 2], out_hbm.at[pl.ds(0, C)], wsems[i % 2]).wait()
            issue_gather(i)

        j = i - 1
        s = j * C
        buf = rows[j % 2]

        @pl.when(s < nv)
        def _gathered():
            pltpu.make_async_copy(
                table_hbm.at[idx_v.at[pl.ds(s, C)]], buf, gsem).wait()

            @pl.when(nv < s + C)
            def _tail():
                def _ztail(k, _):
                    r = lax.shift_right_logical(k, 5)
                    col = jnp.bitwise_and(k, VPR - 1) * 16
                    buf[r, pl.ds(col, 16)] = jnp.zeros((16,), jnp.float32)
                    return 0
                lax.fori_loop((nv - s) * VPR, C * VPR, _ztail, 0)

            pltpu.async_copy(buf, out_hbm.at[pl.ds(base + s, C)], wsems[j % 2])

        @pl.when(nv <= s)
        def _empty():
            pltpu.async_copy(
                zbuf, out_hbm.at[pl.ds(base + s, C)], wsems[j % 2])

    pltpu.make_async_copy(rows0, out_hbm.at[pl.ds(0, C)], wsem0).wait()
    pltpu.make_async_copy(rows1, out_hbm.at[pl.ds(0, C)], wsem1).wait()


@jax.jit
def _sembed(table, ids_flat, lengths):
    mesh = plsc.VectorSubcoreMesh(core_axis_name="c", subcore_axis_name="s")
    f = functools.partial(
        pl.kernel,
        out_type=jax.ShapeDtypeStruct((ROWS, D), jnp.float32),
        mesh=mesh,
        scratch_types=[
            pltpu.VMEM((RPW,), jnp.int32),
            pltpu.VMEM((C, D), jnp.float32),
            pltpu.VMEM((C, D), jnp.float32),
            pltpu.VMEM((C, D), jnp.float32),
            pltpu.VMEM((32,), jnp.int32),
            pltpu.SemaphoreType.DMA,
            pltpu.SemaphoreType.DMA,
            pltpu.SemaphoreType.DMA,
        ],
    )(_body)
    return f(table, ids_flat, lengths)


def kernel(seg_embeddings, edge_index, edge_weights, traj_ids, traj_lengths):
    del edge_index, edge_weights
    out = _sembed(seg_embeddings, traj_ids.reshape(ROWS), traj_lengths)
    return out.reshape(B, L, D)


# speedup vs baseline: 1.5872x; 1.0013x over previous
import functools

import jax
import jax.numpy as jnp
from jax import lax
from jax.experimental import pallas as pl
from jax.experimental.pallas import tpu as pltpu
from jax.experimental.pallas import tpu_sc as plsc

N_SEG, D = 50000, 512
B, L = 16, 2048
ROWS = B * L
NW = 32
C = 64
PCH = L // C
NCHUNK = B
VPR = D // 16


def _body(table_hbm, ids_hbm, len_hbm, out_hbm,
          idx_v, rows0, rows1, zbuf, len_v, gsem, isem, wsem0, wsem1):
    nc = 2
    wid = lax.axis_index("s") * nc + lax.axis_index("c")

    pltpu.sync_copy(len_hbm, len_v.at[pl.ds(0, 16)])

    starts = []
    for j in range(NCHUNK):
        p = jnp.bitwise_and(wid + 2 * j, PCH - 1)
        starts.append(j * L + p * C)

    for j in range(NCHUNK):
        pltpu.async_copy(
            ids_hbm.at[pl.ds(starts[j], C)], idx_v.at[pl.ds(j * C, C)], isem)
    for j in range(NCHUNK):
        pltpu.make_async_copy(
            ids_hbm.at[pl.ds(0, C)], idx_v.at[pl.ds(0, C)], isem).wait()

    lv = len_v[pl.ds(0, 16)]
    nvs = []
    for j in range(NCHUNK):
        nvs.append(jnp.clip(lv[j] - (starts[j] - j * L), 0, C))

    rows = (rows0, rows1)
    wsems = (wsem0, wsem1)

    def issue_gather(i):
        @pl.when(nvs[i] > 0)
        def _():
            pltpu.async_copy(
                table_hbm.at[idx_v.at[pl.ds(i * C, C)]], rows[i % 2], gsem)

    issue_gather(0)

    def _zrow(r, _):
        def _zcol(jj, _):
            zbuf[r, pl.ds(jj * 16, 16)] = jnp.zeros((16,), jnp.float32)
            return 0
        return lax.fori_loop(0, VPR, _zcol, 0)
    lax.fori_loop(0, C, _zrow, 0)

    for i in range(1, NCHUNK + 1):
        if i < NCHUNK:
            if i >= 2:
                pltpu.make_async_copy(
                    rows[i % 2], out_hbm.at[pl.ds(0, C)], wsems[i % 2]).wait()
            issue_gather(i)

        j = i - 1
        buf = rows[j % 2]
        nv = nvs[j]

        @pl.when(nv > 0)
        def _gathered():
            pltpu.make_async_copy(
                table_hbm.at[idx_v.at[pl.ds(j * C, C)]], buf, gsem).wait()

            @pl.when(nv < C)
            def _tail():
                def _ztail(k, _):
                    r = lax.shift_right_logical(k, 5)
                    col = jnp.bitwise_and(k, VPR - 1) * 16
                    buf[r, pl.ds(col, 16)] = jnp.zeros((16,), jnp.float32)
                    return 0
                lax.fori_loop(nv * VPR, C * VPR, _ztail, 0)

            pltpu.async_copy(buf, out_hbm.at[pl.ds(starts[j], C)], wsems[j % 2])

        @pl.when(nv <= 0)
        def _empty():
            pltpu.async_copy(
                zbuf, out_hbm.at[pl.ds(starts[j], C)], wsems[j % 2])

    pltpu.make_async_copy(rows0, out_hbm.at[pl.ds(0, C)], wsem0).wait()
    pltpu.make_async_copy(rows1, out_hbm.at[pl.ds(0, C)], wsem1).wait()


@jax.jit
def _sembed(table, ids_flat, lengths):
    mesh = plsc.VectorSubcoreMesh(core_axis_name="c", subcore_axis_name="s")
    f = functools.partial(
        pl.kernel,
        out_type=jax.ShapeDtypeStruct((ROWS, D), jnp.float32),
        mesh=mesh,
        scratch_types=[
            pltpu.VMEM((NCHUNK * C,), jnp.int32),
            pltpu.VMEM((C, D), jnp.float32),
            pltpu.VMEM((C, D), jnp.float32),
            pltpu.VMEM((C, D), jnp.float32),
            pltpu.VMEM((32,), jnp.int32),
            pltpu.SemaphoreType.DMA,
            pltpu.SemaphoreType.DMA,
            pltpu.SemaphoreType.DMA,
            pltpu.SemaphoreType.DMA,
        ],
    )(_body)
    return f(table, ids_flat, lengths)


def kernel(seg_embeddings, edge_index, edge_weights, traj_ids, traj_lengths):
    del edge_index, edge_weights
    out = _sembed(seg_embeddings, traj_ids.reshape(ROWS), traj_lengths)
    return out.reshape(B, L, D)
